# baseline (speedup 1.0000x reference)
import functools

import jax
import jax.numpy as jnp
from jax import lax
from jax.experimental import pallas as pl
from jax.experimental.pallas import tpu as pltpu
from jax.experimental.pallas import tpu_sc as plsc

D = 128
DW = 64
NUM_WORKERS = 32
CHUNK = 64
NSLOT = 8
LAG = 4


def _make_gather(B):
    b_per_w = B // NUM_WORKERS
    n_chunks = b_per_w // CHUNK
    n_rounds = n_chunks // NSLOT
    mesh = plsc.VectorSubcoreMesh(core_axis_name="c", subcore_axis_name="s")

    @functools.partial(
        pl.kernel,
        mesh=mesh,
        compiler_params=pltpu.CompilerParams(use_tc_tiling_on_sc=False),
        out_type=jax.ShapeDtypeStruct((B, DW), jnp.int32),
        scratch_types=[
            pltpu.VMEM((b_per_w,), jnp.int32),
            pltpu.VMEM((NSLOT, CHUNK, DW), jnp.int32),
        ]
        + [pltpu.SemaphoreType.DMA] * (2 * NSLOT),
    )
    def gather(idx_hbm, table_hbm, out_hbm, idx_v, rows_v, *sems):
        gsem = sems[:NSLOT]
        ssem = sems[NSLOT:]
        wid = lax.axis_index("s") * 2 + lax.axis_index("c")
        base = wid * b_per_w

        pltpu.sync_copy(idx_hbm.at[pl.ds(base, b_per_w)], idx_v)

        def fire_gather(c, slot):
            pltpu.async_copy(
                table_hbm.at[idx_v.at[pl.ds(c * CHUNK, CHUNK)]],
                rows_v.at[slot], gsem[slot])

        def wait_gather(c, slot):
            pltpu.make_async_copy(
                table_hbm.at[idx_v.at[pl.ds(c * CHUNK, CHUNK)]],
                rows_v.at[slot], gsem[slot]).wait()

        def fire_store(c, slot):
            pltpu.async_copy(
                rows_v.at[slot], out_hbm.at[pl.ds(base + c * CHUNK, CHUNK)],
                ssem[slot])

        def wait_store(c, slot):
            pltpu.make_async_copy(
                rows_v.at[slot], out_hbm.at[pl.ds(base + c * CHUNK, CHUNK)],
                ssem[slot]).wait()

        for b in range(NSLOT):
            fire_gather(b, b)
            if b >= LAG:
                wait_gather(b - LAG, b - LAG)
                fire_store(b - LAG, b - LAG)

        def body(r, carry):
            for b in range(NSLOT):
                v = r * NSLOT + b
                wait_store(v - NSLOT, b)
                fire_gather(v, b)
                wait_gather(v - LAG, (b - LAG) % NSLOT)
                fire_store(v - LAG, (b - LAG) % NSLOT)
            return carry

        lax.fori_loop(1, n_rounds, body, 0)

        for c in range(n_chunks - LAG, n_chunks):
            wait_gather(c, c % NSLOT)
            fire_store(c, c % NSLOT)
        for c in range(n_chunks - NSLOT, n_chunks):
            wait_store(c, c % NSLOT)

    return gather


def kernel(input_ids, wte):
    B = input_ids.shape[0] * input_ids.shape[1]
    idx = input_ids.reshape(B)
    v = wte.shape[0]
    table_words = jax.lax.bitcast_convert_type(
        wte.reshape(v, DW, 2), jnp.int32)
    out_words = _make_gather(B)(idx, table_words)
    out = jax.lax.bitcast_convert_type(out_words, jnp.float16).reshape(B, D)
    return out.astype(jnp.float32).reshape(input_ids.shape + (D,))


# speedup vs baseline: 10.3080x; 10.3080x over previous
import functools

import jax
import jax.numpy as jnp
from jax import lax
from jax.experimental import pallas as pl
from jax.experimental.pallas import tpu as pltpu
from jax.experimental.pallas import tpu_sc as plsc

D = 128
NUM_WORKERS = 32
CHUNK = 128
NSLOT = 5
LAG = 2


def _make_gather(B):
    b_per_w = B // NUM_WORKERS
    n_chunks = b_per_w // CHUNK
    n_rounds = n_chunks // NSLOT
    mesh = plsc.VectorSubcoreMesh(core_axis_name="c", subcore_axis_name="s")

    @functools.partial(
        pl.kernel,
        mesh=mesh,
        out_type=jax.ShapeDtypeStruct((B, D), jnp.float32),
        scratch_types=[
            pltpu.VMEM((b_per_w,), jnp.int32),
            pltpu.VMEM((NSLOT, CHUNK, D), jnp.float32),
        ]
        + [pltpu.SemaphoreType.DMA] * (2 * NSLOT),
    )
    def gather(idx_hbm, table_hbm, out_hbm, idx_v, rows_v, *sems):
        gsem = sems[:NSLOT]
        ssem = sems[NSLOT:]
        wid = lax.axis_index("s") * 2 + lax.axis_index("c")
        base = wid * b_per_w

        pltpu.sync_copy(idx_hbm.at[pl.ds(base, b_per_w)], idx_v)

        def fire_gather(c, slot):
            pltpu.async_copy(
                table_hbm.at[idx_v.at[pl.ds(c * CHUNK, CHUNK)]],
                rows_v.at[slot], gsem[slot])

        def wait_gather(c, slot):
            pltpu.make_async_copy(
                table_hbm.at[idx_v.at[pl.ds(c * CHUNK, CHUNK)]],
                rows_v.at[slot], gsem[slot]).wait()

        def fire_store(c, slot):
            pltpu.async_copy(
                rows_v.at[slot], out_hbm.at[pl.ds(base + c * CHUNK, CHUNK)],
                ssem[slot])

        def wait_store(c, slot):
            pltpu.make_async_copy(
                rows_v.at[slot], out_hbm.at[pl.ds(base + c * CHUNK, CHUNK)],
                ssem[slot]).wait()

        for b in range(NSLOT):
            fire_gather(b, b)
            if b >= LAG:
                wait_gather(b - LAG, b - LAG)
                fire_store(b - LAG, b - LAG)

        def body(r, carry):
            for b in range(NSLOT):
                v = r * NSLOT + b
                wait_store(v - NSLOT, b)
                fire_gather(v, b)
                wait_gather(v - LAG, (b - LAG) % NSLOT)
                fire_store(v - LAG, (b - LAG) % NSLOT)
            return carry

        lax.fori_loop(1, n_rounds, body, 0)

        for c in range(n_chunks - LAG, n_chunks):
            wait_gather(c, c % NSLOT)
            fire_store(c, c % NSLOT)
        for c in range(n_chunks - NSLOT, n_chunks):
            wait_store(c, c % NSLOT)

    return gather


def kernel(input_ids, wte):
    B = input_ids.shape[0] * input_ids.shape[1]
    idx = input_ids.reshape(B)
    table = wte.astype(jnp.float32)
    out = _make_gather(B)(idx, table)
    return out.reshape(input_ids.shape + (D,))


# speedup vs baseline: 10.3338x; 1.0025x over previous
import functools

import jax
import jax.numpy as jnp
from jax import lax
from jax.experimental import pallas as pl
from jax.experimental.pallas import tpu as pltpu
from jax.experimental.pallas import tpu_sc as plsc

D = 128
NUM_WORKERS = 32
CHUNK = 128
NSLOT = 4
LAG = 2


def _make_gather(B):
    b_per_w = B // NUM_WORKERS
    n_chunks = b_per_w // CHUNK
    n_rounds = n_chunks // NSLOT
    mesh = plsc.VectorSubcoreMesh(core_axis_name="c", subcore_axis_name="s")

    @functools.partial(
        pl.kernel,
        mesh=mesh,
        out_type=jax.ShapeDtypeStruct((B, D), jnp.float32),
        scratch_types=[
            pltpu.VMEM((b_per_w,), jnp.int32),
            pltpu.VMEM((NSLOT, CHUNK, D), jnp.float32),
        ]
        + [pltpu.SemaphoreType.DMA] * (2 * NSLOT),
    )
    def gather(idx_hbm, table_hbm, out_hbm, idx_v, rows_v, *sems):
        gsem = sems[:NSLOT]
        ssem = sems[NSLOT:]
        wid = lax.axis_index("s") * 2 + lax.axis_index("c")
        base = wid * b_per_w

        pltpu.sync_copy(idx_hbm.at[pl.ds(base, b_per_w)], idx_v)

        def fire_gather(c, slot):
            pltpu.async_copy(
                table_hbm.at[idx_v.at[pl.ds(c * CHUNK, CHUNK)]],
                rows_v.at[slot], gsem[slot])

        def wait_gather(c, slot):
            pltpu.make_async_copy(
                table_hbm.at[idx_v.at[pl.ds(c * CHUNK, CHUNK)]],
                rows_v.at[slot], gsem[slot]).wait()

        def fire_store(c, slot):
            pltpu.async_copy(
                rows_v.at[slot], out_hbm.at[pl.ds(base + c * CHUNK, CHUNK)],
                ssem[slot])

        def wait_store(c, slot):
            pltpu.make_async_copy(
                rows_v.at[slot], out_hbm.at[pl.ds(base + c * CHUNK, CHUNK)],
                ssem[slot]).wait()

        for b in range(NSLOT):
            fire_gather(b, b)
            if b >= LAG:
                wait_gather(b - LAG, b - LAG)
                fire_store(b - LAG, b - LAG)

        def body(r, carry):
            for b in range(NSLOT):
                v = r * NSLOT + b
                wait_store(v - NSLOT, b)
                fire_gather(v, b)
                wait_gather(v - LAG, (b - LAG) % NSLOT)
                fire_store(v - LAG, (b - LAG) % NSLOT)
            return carry

        lax.fori_loop(1, n_rounds, body, 0)

        for c in range(n_chunks - LAG, n_chunks):
            wait_gather(c, c % NSLOT)
            fire_store(c, c % NSLOT)
        for c in range(n_chunks - NSLOT, n_chunks):
            wait_store(c, c % NSLOT)

    return gather


def kernel(input_ids, wte):
    B = input_ids.shape[0] * input_ids.shape[1]
    idx = input_ids.reshape(B)
    table = wte.astype(jnp.float32)
    out = _make_gather(B)(idx, table)
    return out.reshape(input_ids.shape + (D,))


# speedup vs baseline: 10.3836x; 1.0048x over previous
import functools

import jax
import jax.numpy as jnp
from jax import lax
from jax.experimental import pallas as pl
from jax.experimental.pallas import tpu as pltpu
from jax.experimental.pallas import tpu_sc as plsc

D = 128
NUM_WORKERS = 32
CHUNK = 128
NBUF = 4


def _make_gather(B):
    b_per_w = B // NUM_WORKERS
    n_chunks = b_per_w // CHUNK
    n_rounds = n_chunks // NBUF
    mesh = plsc.VectorSubcoreMesh(core_axis_name="c", subcore_axis_name="s")

    @functools.partial(
        pl.kernel,
        mesh=mesh,
        out_type=jax.ShapeDtypeStruct((B, D), jnp.float32),
        scratch_types=[
            pltpu.VMEM((NBUF, CHUNK), jnp.int32),
            pltpu.VMEM((NBUF, CHUNK, D), jnp.float32),
        ]
        + [pltpu.SemaphoreType.DMA] * (2 * NBUF),
    )
    def gather(idx_hbm, table_hbm, out_hbm, idx_v, rows_v, *sems):
        gsem = sems[:NBUF]
        ssem = sems[NBUF:]
        wid = lax.axis_index("s") * 2 + lax.axis_index("c")
        base = wid * b_per_w

        for b in range(NBUF):
            pltpu.sync_copy(idx_hbm.at[pl.ds(base + b * CHUNK, CHUNK)],
                            idx_v.at[b])
            pltpu.async_copy(table_hbm.at[idx_v.at[b]], rows_v.at[b], gsem[b])

        def body(r, carry):
            for b in range(NBUF):
                c_off = base + (r * NBUF + b) * CHUNK
                pltpu.make_async_copy(table_hbm.at[idx_v.at[b]],
                                      rows_v.at[b], gsem[b]).wait()
                pltpu.async_copy(rows_v.at[b],
                                 out_hbm.at[pl.ds(c_off, CHUNK)], ssem[b])

                @pl.when(r < n_rounds - 1)
                def _():
                    n_off = c_off + NBUF * CHUNK
                    pltpu.sync_copy(idx_hbm.at[pl.ds(n_off, CHUNK)],
                                    idx_v.at[b])
                    pltpu.make_async_copy(rows_v.at[b],
                                          out_hbm.at[pl.ds(c_off, CHUNK)],
                                          ssem[b]).wait()
                    pltpu.async_copy(table_hbm.at[idx_v.at[b]],
                                     rows_v.at[b], gsem[b])
            return carry

        lax.fori_loop(0, n_rounds, body, 0)

        last = base + (n_chunks - NBUF) * CHUNK
        for b in range(NBUF):
            pltpu.make_async_copy(
                rows_v.at[b],
                out_hbm.at[pl.ds(last + b * CHUNK, CHUNK)], ssem[b]).wait()

    return gather


def kernel(input_ids, wte):
    B = input_ids.shape[0] * input_ids.shape[1]
    idx = input_ids.reshape(B)
    table = wte.astype(jnp.float32)
    out = _make_gather(B)(idx, table)
    return out.reshape(input_ids.shape + (D,))
